# baseline (speedup 1.0000x reference)
import functools
import math

import jax
import jax.numpy as jnp
from jax import lax
from jax.experimental import pallas as pl
from jax.experimental.pallas import tpu as pltpu
from jax.experimental.pallas import tpu_sc as plsc

TOP_K = 2
CAP_FACTOR = 1.25
LANES = 128


def _router(flat, gwt_pad, N, E, CAP, TB):
    nblk = N // TB
    S_TOT = E * CAP
    f32 = jnp.float32

    def body(x_ref, gwt_ref, kept_ref, dst_ref, src_ref, gate_ref,
             carry, src_scr, gate_scr, fil_scr, dst_scr):
        gi = pl.program_id(0)

        @pl.when(gi == 0)
        def _init():
            carry[...] = jnp.zeros_like(carry)
            src_scr[...] = jnp.zeros_like(src_scr)
            gate_scr[...] = jnp.zeros_like(gate_scr)
            fil_scr[...] = jnp.zeros_like(fil_scr)
            dst_scr[...] = jnp.zeros_like(dst_scr)

        x = x_ref[...]
        logits = jnp.dot(x, gwt_ref[...], preferred_element_type=f32)
        lanef = lax.broadcasted_iota(jnp.int32, (TB, LANES), 1).astype(f32)
        logits = jnp.where(lanef < float(E), logits, -1e30)
        m = jnp.max(logits, axis=1, keepdims=True)
        ex = jnp.exp(logits - m)
        gates = ex / jnp.sum(ex, axis=1, keepdims=True)

        v1 = jnp.max(gates, axis=1, keepdims=True)
        i1 = jnp.min(jnp.where(gates == v1, lanef, 1e9), axis=1, keepdims=True)
        g2m = jnp.where(lanef == i1, -1.0, gates)
        v2 = jnp.max(g2m, axis=1, keepdims=True)
        i2 = jnp.min(jnp.where(g2m == v2, lanef, 1e9), axis=1, keepdims=True)

        maskf = ((lanef == i1) | (lanef == i2)).astype(f32)
        tri = (lax.broadcasted_iota(jnp.int32, (TB, TB), 0)
               >= lax.broadcasted_iota(jnp.int32, (TB, TB), 1)).astype(f32)
        csum = jnp.dot(tri, maskf, preferred_element_type=f32)
        posm = csum + carry[...] - 1.0
        carry[...] = carry[...] + csum[TB - 1:TB, :]

        pos1 = jnp.sum(jnp.where(lanef == i1, posm, 0.0), axis=1, keepdims=True)
        pos2 = jnp.sum(jnp.where(lanef == i2, posm, 0.0), axis=1, keepdims=True)
        kept1 = pos1 < float(CAP)
        kept2 = pos2 < float(CAP)
        s1 = i1 * float(CAP) + pos1
        s2 = i2 * float(CAP) + pos2

        kept_ref[...] = jnp.concatenate(
            [kept1.astype(f32), kept2.astype(f32)], axis=1).reshape(1, TB, 2)

        lane_s = lax.broadcasted_iota(jnp.int32, (TB, S_TOT), 1).astype(f32)
        oh1 = (lane_s == jnp.where(kept1, s1, -1.0)).astype(f32)
        oh2 = (lane_s == jnp.where(kept2, s2, -1.0)).astype(f32)
        ohall = oh1 + oh2

        tokcol = float(TB) * gi + lax.broadcasted_iota(jnp.int32, (TB, 1), 0).astype(f32)
        lhsA = jnp.concatenate([tokcol, jnp.ones((TB, 1), f32)], axis=1)
        dnum = (((0,), (0,)), ((), ()))
        accA = lax.dot_general(lhsA, ohall, dnum,
                               precision=lax.Precision.HIGHEST,
                               preferred_element_type=f32)
        src_scr[...] += accA[0:1, :]
        fil_scr[...] += accA[1:2, :]
        accB1 = lax.dot_general(jnp.concatenate([tokcol, v1], axis=1), oh1,
                                dnum, precision=lax.Precision.HIGHEST,
                                preferred_element_type=f32)
        accB2 = lax.dot_general(
            jnp.concatenate([tokcol + float(N + 8), v2], axis=1), oh2,
            dnum, precision=lax.Precision.HIGHEST, preferred_element_type=f32)
        dst_scr[...] += accB1[0:1, :] + accB2[0:1, :]
        gate_scr[...] += accB1[1:2, :] + accB2[1:2, :]

        @pl.when(gi == nblk - 1)
        def _fin():
            src_ref[...] = jnp.where(fil_scr[...] > 0.5,
                                     jnp.round(src_scr[...]),
                                     float(N)).astype(jnp.int32)
            dst_ref[...] = jnp.where(fil_scr[...] > 0.5,
                                     jnp.round(dst_scr[...]),
                                     float(N)).astype(jnp.int32)
            gate_ref[...] = gate_scr[...]

    out_shapes = (
        jax.ShapeDtypeStruct((nblk, TB, 2), f32),
        jax.ShapeDtypeStruct((1, S_TOT), jnp.int32),
        jax.ShapeDtypeStruct((1, S_TOT), jnp.int32),
        jax.ShapeDtypeStruct((1, S_TOT), f32),
    )
    return pl.pallas_call(
        body,
        grid=(nblk,),
        in_specs=[
            pl.BlockSpec((TB, flat.shape[1]), lambda g: (g, 0)),
            pl.BlockSpec((flat.shape[1], LANES), lambda g: (0, 0)),
        ],
        out_specs=(
            pl.BlockSpec((1, TB, 2), lambda g: (g, 0, 0)),
            pl.BlockSpec((1, S_TOT), lambda g: (0, 0)),
            pl.BlockSpec((1, S_TOT), lambda g: (0, 0)),
            pl.BlockSpec((1, S_TOT), lambda g: (0, 0)),
        ),
        out_shape=out_shapes,
        scratch_shapes=[
            pltpu.VMEM((1, LANES), f32),
            pltpu.VMEM((1, S_TOT), f32),
            pltpu.VMEM((1, S_TOT), f32),
            pltpu.VMEM((1, S_TOT), f32),
            pltpu.VMEM((1, S_TOT), f32),
        ],
        compiler_params=pltpu.CompilerParams(
            dimension_semantics=("arbitrary",)),
    )(flat, gwt_pad)


def _make_dispatch(V, D, S_TOT, NC, NS):
    NW = NC * NS
    rows_per = S_TOT // NW
    mesh = plsc.VectorSubcoreMesh(core_axis_name="c", subcore_axis_name="s")

    @functools.partial(
        pl.kernel, mesh=mesh,
        out_type=jax.ShapeDtypeStruct((S_TOT, D), jnp.float32),
        scratch_types=[
            pltpu.VMEM((rows_per,), jnp.int32),
            pltpu.VMEM((rows_per, D), jnp.float32),
            pltpu.SemaphoreType.DMA,
        ],
    )
    def dispatch_k(table_hbm, idx_hbm, out_hbm, idx_v, rows_v, sem):
        wid = lax.axis_index("s") * NC + lax.axis_index("c")
        base = wid * rows_per
        pltpu.sync_copy(idx_hbm.at[pl.ds(base, rows_per)], idx_v)
        pltpu.async_copy(table_hbm.at[idx_v], rows_v, sem).wait()
        pltpu.sync_copy(rows_v, out_hbm.at[pl.ds(base, rows_per)])

    return dispatch_k


def _make_combine(N, D, S_TOT, NC, NS):
    NW = NC * NS
    rows_per = S_TOT // NW
    mesh = plsc.VectorSubcoreMesh(core_axis_name="c", subcore_axis_name="s")

    @functools.partial(
        pl.kernel, mesh=mesh,
        out_type=jax.ShapeDtypeStruct((2 * (N + 8), D), jnp.float32),
        scratch_types=[
            pltpu.VMEM((rows_per,), jnp.int32),
            pltpu.VMEM((rows_per, D), jnp.float32),
            pltpu.SemaphoreType.DMA,
        ],
    )
    def combine_k(rows_hbm, idx_hbm, out_hbm, idx_v, buf_v, sem):
        wid = lax.axis_index("s") * NC + lax.axis_index("c")
        base = wid * rows_per
        pltpu.sync_copy(idx_hbm.at[pl.ds(base, rows_per)], idx_v)
        pltpu.sync_copy(rows_hbm.at[pl.ds(base, rows_per)], buf_v)
        pltpu.async_copy(buf_v, out_hbm.at[idx_v], sem).wait()

    return combine_k


def _add_halves(rows2, kept, N, D, TB):
    def body(a_ref, b_ref, k_ref, o_ref):
        a = jnp.where(k_ref[:, 0:1] > 0.5, a_ref[0], 0.0)
        b = jnp.where(k_ref[:, 1:2] > 0.5, b_ref[0], 0.0)
        o_ref[...] = a + b

    return pl.pallas_call(
        body,
        grid=(N // TB,),
        in_specs=[pl.BlockSpec((1, TB, D), lambda g: (0, g, 0)),
                  pl.BlockSpec((1, TB, D), lambda g: (1, g, 0)),
                  pl.BlockSpec((TB, 2), lambda g: (g, 0))],
        out_specs=pl.BlockSpec((TB, D), lambda g: (g, 0)),
        out_shape=jax.ShapeDtypeStruct((N, D), jnp.float32),
    )(rows2, rows2, kept)


def _ffn(xin, W1, b1, W2, b2, gate_col, E, CAP, D, H, HBK):
    nh = H // HBK
    f32 = jnp.float32

    def body(x_ref, w1_ref, b1_ref, w2_ref, b2_ref, g_ref, out_ref, acc):
        hi = pl.program_id(1)
        bf16 = jnp.bfloat16
        x = x_ref[0].astype(bf16)
        hpre = lax.dot_general(x, w1_ref[0].astype(bf16),
                               (((1,), (1,)), ((), ())),
                               preferred_element_type=f32) + b1_ref[0]
        hact = 0.5 * hpre * (1.0 + lax.erf(hpre * (1.0 / math.sqrt(2.0))))
        part = lax.dot_general(hact.astype(bf16), w2_ref[0].astype(bf16),
                               (((1,), (1,)), ((), ())),
                               preferred_element_type=f32)

        @pl.when(hi == 0)
        def _z():
            acc[...] = jnp.zeros_like(acc)

        acc[...] += part

        @pl.when(hi == nh - 1)
        def _w():
            out_ref[...] = ((acc[...] + b2_ref[0]) * g_ref[0]
                            ).reshape(1, CAP, D)

    return pl.pallas_call(
        body,
        grid=(E, nh),
        in_specs=[
            pl.BlockSpec((1, CAP, D), lambda e, h: (e, 0, 0)),
            pl.BlockSpec((1, HBK, D), lambda e, h: (e, h, 0)),
            pl.BlockSpec((1, 1, HBK), lambda e, h: (e, 0, h)),
            pl.BlockSpec((1, D, HBK), lambda e, h: (e, 0, h)),
            pl.BlockSpec((1, 1, D), lambda e, h: (e, 0, 0)),
            pl.BlockSpec((1, CAP, 1), lambda e, h: (e, 0, 0)),
        ],
        out_specs=pl.BlockSpec((1, CAP, D), lambda e, h: (e, 0, 0)),
        out_shape=jax.ShapeDtypeStruct((E, CAP, D), f32),
        scratch_shapes=[pltpu.VMEM((CAP, D), f32)],
        compiler_params=pltpu.CompilerParams(
            dimension_semantics=("arbitrary", "arbitrary")),
    )(xin, W1, b1.reshape(E, 1, H), W2, b2.reshape(E, 1, D), gate_col)


def kernel(hidden_states, gate_W, W1, b1, W2, b2):
    Bh, Sh, D = hidden_states.shape
    N = Bh * Sh
    E = gate_W.shape[0]
    H = W1.shape[1]
    CAP = max(1, math.ceil(CAP_FACTOR * N / E))
    S_TOT = E * CAP
    TB = 256
    f32 = jnp.float32

    info = plsc.get_sparse_core_info()
    NC, NS = info.num_cores, info.num_subcores

    flat = hidden_states.reshape(N, D).astype(f32)
    gwt_pad = jnp.pad(gate_W.astype(f32).T, ((0, 0), (0, LANES - E)))

    kept, dst_idx, src_idx, gate_slot = _router(flat, gwt_pad, N, E, CAP, TB)

    flat_pad = jnp.concatenate([flat, jnp.zeros((8, D), f32)], axis=0)
    dispatch_k = _make_dispatch(flat_pad.shape[0], D, S_TOT, NC, NS)
    xin = dispatch_k(flat_pad, src_idx.reshape(S_TOT))

    outg = _ffn(xin.reshape(E, CAP, D), W1.astype(f32), b1.astype(f32),
                W2.astype(f32), b2.astype(f32),
                gate_slot.reshape(E, CAP, 1), E, CAP, D, H, 512)

    combine_k = _make_combine(N, D, S_TOT, NC, NS)
    rows2 = combine_k(outg.reshape(S_TOT, D), dst_idx.reshape(S_TOT))
    final = _add_halves(rows2.reshape(2, N + 8, D), kept.reshape(N, 2),
                        N, D, TB)

    aux = jnp.zeros((), hidden_states.dtype)
    return final.reshape(Bh, Sh, D), aux


# speedup vs baseline: 1.0359x; 1.0359x over previous
import functools
import math

import jax
import jax.numpy as jnp
from jax import lax
from jax.experimental import pallas as pl
from jax.experimental.pallas import tpu as pltpu
from jax.experimental.pallas import tpu_sc as plsc

TOP_K = 2
CAP_FACTOR = 1.25
LANES = 128


def _router(flat, gwt_pad, N, E, CAP, TB):
    nblk = N // TB
    S_TOT = E * CAP
    f32 = jnp.float32

    def body(x_ref, gwt_ref, kept_ref, dst_ref, src_ref, gate_ref,
             carry, src_scr, gate_scr, fil_scr, dst_scr):
        gi = pl.program_id(0)

        @pl.when(gi == 0)
        def _init():
            carry[...] = jnp.zeros_like(carry)
            src_scr[...] = jnp.zeros_like(src_scr)
            gate_scr[...] = jnp.zeros_like(gate_scr)
            fil_scr[...] = jnp.zeros_like(fil_scr)
            dst_scr[...] = jnp.zeros_like(dst_scr)

        x = x_ref[...]
        logits = jnp.dot(x, gwt_ref[...], preferred_element_type=f32)
        lanef = lax.broadcasted_iota(jnp.int32, (TB, LANES), 1).astype(f32)
        logits = jnp.where(lanef < float(E), logits, -1e30)
        m = jnp.max(logits, axis=1, keepdims=True)
        ex = jnp.exp(logits - m)
        gates = ex / jnp.sum(ex, axis=1, keepdims=True)

        v1 = jnp.max(gates, axis=1, keepdims=True)
        i1 = jnp.min(jnp.where(gates == v1, lanef, 1e9), axis=1, keepdims=True)
        g2m = jnp.where(lanef == i1, -1.0, gates)
        v2 = jnp.max(g2m, axis=1, keepdims=True)
        i2 = jnp.min(jnp.where(g2m == v2, lanef, 1e9), axis=1, keepdims=True)

        maskf = ((lanef == i1) | (lanef == i2)).astype(f32)
        tri = (lax.broadcasted_iota(jnp.int32, (TB, TB), 0)
               >= lax.broadcasted_iota(jnp.int32, (TB, TB), 1)).astype(f32)
        csum = jnp.dot(tri, maskf, preferred_element_type=f32)
        posm = csum + carry[...] - 1.0
        carry[...] = carry[...] + csum[TB - 1:TB, :]

        pos1 = jnp.sum(jnp.where(lanef == i1, posm, 0.0), axis=1, keepdims=True)
        pos2 = jnp.sum(jnp.where(lanef == i2, posm, 0.0), axis=1, keepdims=True)
        kept1 = pos1 < float(CAP)
        kept2 = pos2 < float(CAP)
        s1 = i1 * float(CAP) + pos1
        s2 = i2 * float(CAP) + pos2

        kept_ref[...] = jnp.concatenate(
            [kept1.astype(f32), kept2.astype(f32)], axis=1).reshape(1, TB, 2)

        lane_s = lax.broadcasted_iota(jnp.int32, (TB, S_TOT), 1).astype(f32)
        oh1 = (lane_s == jnp.where(kept1, s1, -1.0)).astype(f32)
        oh2 = (lane_s == jnp.where(kept2, s2, -1.0)).astype(f32)
        ohall = oh1 + oh2

        tokcol = float(TB) * gi + lax.broadcasted_iota(jnp.int32, (TB, 1), 0).astype(f32)
        lhsA = jnp.concatenate([tokcol, jnp.ones((TB, 1), f32)], axis=1)
        dnum = (((0,), (0,)), ((), ()))
        accA = lax.dot_general(lhsA, ohall, dnum,
                               precision=lax.Precision.HIGHEST,
                               preferred_element_type=f32)
        src_scr[...] += accA[0:1, :]
        fil_scr[...] += accA[1:2, :]
        accB1 = lax.dot_general(jnp.concatenate([tokcol, v1], axis=1), oh1,
                                dnum, precision=lax.Precision.HIGHEST,
                                preferred_element_type=f32)
        accB2 = lax.dot_general(
            jnp.concatenate([tokcol + float(N + 8), v2], axis=1), oh2,
            dnum, precision=lax.Precision.HIGHEST, preferred_element_type=f32)
        dst_scr[...] += accB1[0:1, :] + accB2[0:1, :]
        gate_scr[...] += accB1[1:2, :] + accB2[1:2, :]

        @pl.when(gi == nblk - 1)
        def _fin():
            src_ref[...] = jnp.where(fil_scr[...] > 0.5,
                                     jnp.round(src_scr[...]),
                                     0.0).astype(jnp.int32)
            dst_ref[...] = jnp.where(fil_scr[...] > 0.5,
                                     jnp.round(dst_scr[...]),
                                     float(N)).astype(jnp.int32)
            gate_ref[...] = gate_scr[...]

    out_shapes = (
        jax.ShapeDtypeStruct((nblk, TB, 2), f32),
        jax.ShapeDtypeStruct((1, S_TOT), jnp.int32),
        jax.ShapeDtypeStruct((1, S_TOT), jnp.int32),
        jax.ShapeDtypeStruct((1, S_TOT), f32),
    )
    return pl.pallas_call(
        body,
        grid=(nblk,),
        in_specs=[
            pl.BlockSpec((TB, flat.shape[1]), lambda g: (g, 0)),
            pl.BlockSpec((flat.shape[1], LANES), lambda g: (0, 0)),
        ],
        out_specs=(
            pl.BlockSpec((1, TB, 2), lambda g: (g, 0, 0)),
            pl.BlockSpec((1, S_TOT), lambda g: (0, 0)),
            pl.BlockSpec((1, S_TOT), lambda g: (0, 0)),
            pl.BlockSpec((1, S_TOT), lambda g: (0, 0)),
        ),
        out_shape=out_shapes,
        scratch_shapes=[
            pltpu.VMEM((1, LANES), f32),
            pltpu.VMEM((1, S_TOT), f32),
            pltpu.VMEM((1, S_TOT), f32),
            pltpu.VMEM((1, S_TOT), f32),
            pltpu.VMEM((1, S_TOT), f32),
        ],
        compiler_params=pltpu.CompilerParams(
            dimension_semantics=("arbitrary",)),
    )(flat, gwt_pad)


def _make_dispatch(V, D, S_TOT, NC, NS):
    NW = NC * NS
    rows_per = S_TOT // NW
    mesh = plsc.VectorSubcoreMesh(core_axis_name="c", subcore_axis_name="s")

    @functools.partial(
        pl.kernel, mesh=mesh,
        out_type=jax.ShapeDtypeStruct((S_TOT, D), jnp.float32),
        scratch_types=[
            pltpu.VMEM((rows_per,), jnp.int32),
            pltpu.VMEM((rows_per, D), jnp.float32),
            pltpu.SemaphoreType.DMA,
        ],
    )
    def dispatch_k(table_hbm, idx_hbm, out_hbm, idx_v, rows_v, sem):
        wid = lax.axis_index("s") * NC + lax.axis_index("c")
        base = wid * rows_per
        pltpu.sync_copy(idx_hbm.at[pl.ds(base, rows_per)], idx_v)
        pltpu.async_copy(table_hbm.at[idx_v], rows_v, sem).wait()
        pltpu.sync_copy(rows_v, out_hbm.at[pl.ds(base, rows_per)])

    return dispatch_k


def _make_combine(N, D, S_TOT, NC, NS):
    NW = NC * NS
    rows_per = S_TOT // NW
    mesh = plsc.VectorSubcoreMesh(core_axis_name="c", subcore_axis_name="s")

    @functools.partial(
        pl.kernel, mesh=mesh,
        out_type=jax.ShapeDtypeStruct((2 * (N + 8), D), jnp.float32),
        scratch_types=[
            pltpu.VMEM((rows_per,), jnp.int32),
            pltpu.VMEM((rows_per, D), jnp.float32),
            pltpu.SemaphoreType.DMA,
        ],
    )
    def combine_k(rows_hbm, idx_hbm, out_hbm, idx_v, buf_v, sem):
        wid = lax.axis_index("s") * NC + lax.axis_index("c")
        base = wid * rows_per
        pltpu.sync_copy(idx_hbm.at[pl.ds(base, rows_per)], idx_v)
        pltpu.sync_copy(rows_hbm.at[pl.ds(base, rows_per)], buf_v)
        pltpu.async_copy(buf_v, out_hbm.at[idx_v], sem).wait()

    return combine_k


def _add_halves(rows2, kept, N, D, TB):
    def body(a_ref, b_ref, k_ref, o_ref):
        a = jnp.where(k_ref[:, 0:1] > 0.5, a_ref[0], 0.0)
        b = jnp.where(k_ref[:, 1:2] > 0.5, b_ref[0], 0.0)
        o_ref[...] = a + b

    return pl.pallas_call(
        body,
        grid=(N // TB,),
        in_specs=[pl.BlockSpec((1, TB, D), lambda g: (0, g, 0)),
                  pl.BlockSpec((1, TB, D), lambda g: (1, g, 0)),
                  pl.BlockSpec((TB, 2), lambda g: (g, 0))],
        out_specs=pl.BlockSpec((TB, D), lambda g: (g, 0)),
        out_shape=jax.ShapeDtypeStruct((N, D), jnp.float32),
    )(rows2, rows2, kept)


def _ffn(xin, W1, b1, W2, b2, gate_col, E, CAP, D, H, HBK):
    nh = H // HBK
    f32 = jnp.float32

    def body(x_ref, w1_ref, b1_ref, w2_ref, b2_ref, g_ref, out_ref, acc):
        hi = pl.program_id(1)
        bf16 = jnp.bfloat16
        x = x_ref[0].astype(bf16)
        hpre = lax.dot_general(x, w1_ref[0].astype(bf16),
                               (((1,), (1,)), ((), ())),
                               preferred_element_type=f32) + b1_ref[0]
        hact = 0.5 * hpre * (1.0 + lax.erf(hpre * (1.0 / math.sqrt(2.0))))
        part = lax.dot_general(hact.astype(bf16), w2_ref[0].astype(bf16),
                               (((1,), (1,)), ((), ())),
                               preferred_element_type=f32)

        @pl.when(hi == 0)
        def _z():
            acc[...] = jnp.zeros_like(acc)

        acc[...] += part

        @pl.when(hi == nh - 1)
        def _w():
            out_ref[...] = ((acc[...] + b2_ref[0]) * g_ref[0]
                            ).reshape(1, CAP, D)

    return pl.pallas_call(
        body,
        grid=(E, nh),
        in_specs=[
            pl.BlockSpec((1, CAP, D), lambda e, h: (e, 0, 0)),
            pl.BlockSpec((1, HBK, D), lambda e, h: (e, h, 0)),
            pl.BlockSpec((1, 1, HBK), lambda e, h: (e, 0, h)),
            pl.BlockSpec((1, D, HBK), lambda e, h: (e, 0, h)),
            pl.BlockSpec((1, 1, D), lambda e, h: (e, 0, 0)),
            pl.BlockSpec((1, CAP, 1), lambda e, h: (e, 0, 0)),
        ],
        out_specs=pl.BlockSpec((1, CAP, D), lambda e, h: (e, 0, 0)),
        out_shape=jax.ShapeDtypeStruct((E, CAP, D), f32),
        scratch_shapes=[pltpu.VMEM((CAP, D), f32)],
        compiler_params=pltpu.CompilerParams(
            dimension_semantics=("arbitrary", "arbitrary")),
    )(xin, W1, b1.reshape(E, 1, H), W2, b2.reshape(E, 1, D), gate_col)


def kernel(hidden_states, gate_W, W1, b1, W2, b2):
    Bh, Sh, D = hidden_states.shape
    N = Bh * Sh
    E = gate_W.shape[0]
    H = W1.shape[1]
    CAP = max(1, math.ceil(CAP_FACTOR * N / E))
    S_TOT = E * CAP
    TB = 256
    f32 = jnp.float32

    info = plsc.get_sparse_core_info()
    NC, NS = info.num_cores, info.num_subcores

    flat = hidden_states.reshape(N, D).astype(f32)
    gwt_pad = jnp.pad(gate_W.astype(f32).T, ((0, 0), (0, LANES - E)))

    kept, dst_idx, src_idx, gate_slot = _router(flat, gwt_pad, N, E, CAP, TB)

    dispatch_k = _make_dispatch(N, D, S_TOT, NC, NS)
    xin = dispatch_k(flat, src_idx.reshape(S_TOT))

    outg = _ffn(xin.reshape(E, CAP, D), W1.astype(f32), b1.astype(f32),
                W2.astype(f32), b2.astype(f32),
                gate_slot.reshape(E, CAP, 1), E, CAP, D, H, 512)

    combine_k = _make_combine(N, D, S_TOT, NC, NS)
    rows2 = combine_k(outg.reshape(S_TOT, D), dst_idx.reshape(S_TOT))
    final = _add_halves(rows2.reshape(2, N + 8, D), kept.reshape(N, 2),
                        N, D, TB)

    aux = jnp.zeros((), hidden_states.dtype)
    return final.reshape(Bh, Sh, D), aux
